# baseline (speedup 1.0000x reference)
import functools

import jax
import jax.numpy as jnp
from jax import lax
from jax.experimental import pallas as pl
from jax.experimental.pallas import tpu as pltpu
from jax.experimental.pallas import tpu_sc as plsc

B = 8192
IN_FEATURES = 2048
N0, N1, N2, N3 = 2048, 1024, 512, 1024
NUM_T = 16
BLK = 512
G = 2 * B // BLK + NUM_T + 1
RP = G * BLK

_SC_CORES = 2
_SC_SUBCORES = 16
_NW = _SC_CORES * _SC_SUBCORES


def _pack_pairs(x_bf16):
    n = x_bf16.shape[-1]
    lo = lax.bitcast_convert_type(x_bf16[:, :n // 2],
                                  jnp.uint16).astype(jnp.uint32)
    hi = lax.bitcast_convert_type(x_bf16[:, n // 2:],
                                  jnp.uint16).astype(jnp.uint32)
    return lax.bitcast_convert_type(lo | (hi << 16), jnp.float32)


def _unpack_halves(p_f32):
    u = lax.bitcast_convert_type(p_f32, jnp.uint32)
    lo = lax.bitcast_convert_type((u & 0xFFFF).astype(jnp.uint16),
                                  jnp.bfloat16)
    hi = lax.bitcast_convert_type((u >> 16).astype(jnp.uint16),
                                  jnp.bfloat16)
    return lo, hi


def _enc_body(x_ref, w1_ref, b1_ref, w2_ref, b2_ref, o_ref, w1b, w2b):
    @pl.when(pl.program_id(0) == 0)
    def _():
        w1b[...] = w1_ref[...].astype(jnp.bfloat16)
        w2b[...] = w2_ref[...].astype(jnp.bfloat16)

    x = x_ref[...].astype(jnp.bfloat16)
    h = jnp.dot(x, w1b[...], preferred_element_type=jnp.float32)
    h = jnp.maximum(h + b1_ref[...], 0.0).astype(jnp.bfloat16)
    e = jnp.dot(h, w2b[...], preferred_element_type=jnp.float32)
    e = jnp.maximum(e + b2_ref[...], 0.0)
    o_ref[...] = _pack_pairs(e.astype(jnp.bfloat16))


def _encoder(x, w1, b1, w2, b2):
    bm = 512
    return pl.pallas_call(
        _enc_body,
        grid=(B // bm,),
        in_specs=[
            pl.BlockSpec((bm, IN_FEATURES), lambda i: (i, 0)),
            pl.BlockSpec((IN_FEATURES, N0), lambda i: (0, 0)),
            pl.BlockSpec((1, N0), lambda i: (0, 0)),
            pl.BlockSpec((N0, N1), lambda i: (0, 0)),
            pl.BlockSpec((1, N1), lambda i: (0, 0)),
        ],
        out_specs=pl.BlockSpec((bm, N1 // 2), lambda i: (i, 0)),
        out_shape=jax.ShapeDtypeStruct((B, N1 // 2), jnp.float32),
        scratch_shapes=[
            pltpu.VMEM((IN_FEATURES, N0), jnp.bfloat16),
            pltpu.VMEM((N0, N1), jnp.bfloat16),
        ],
    )(x, w1, b1, w2, b2)


def _sc_scatter_rows(src, idx, n_out, d, chunk):
    n_pairs = idx.shape[0]
    src_rows = src.shape[0]
    rpw = n_pairs // _NW
    mesh = plsc.VectorSubcoreMesh(core_axis_name="c", subcore_axis_name="s")

    n = rpw // chunk

    @functools.partial(
        pl.kernel,
        mesh=mesh,
        out_type=jax.ShapeDtypeStruct((n_out, d), jnp.float32),
        scratch_types=[
            pltpu.VMEM((n, chunk), jnp.int32),
            pltpu.VMEM((2, chunk, d), jnp.float32),
            pltpu.SemaphoreType.DMA,
            pltpu.SemaphoreType.DMA,
        ],
    )
    def sk(src_hbm, idx_hbm, out_hbm, idx2, bufs, wsem0, wsem1):
        wid = lax.axis_index("s") * _SC_CORES + lax.axis_index("c")
        base = wid * rpw
        src_base = base % src_rows
        for j in range(n):
            pltpu.sync_copy(idx_hbm.at[pl.ds(base + j * chunk, chunk)],
                            idx2.at[j])
        wsems = (wsem0, wsem1)
        pend = [None, None]
        for j in range(n):
            s = j % 2
            if pend[s] is not None:
                pend[s].wait()
            pltpu.sync_copy(src_hbm.at[pl.ds(src_base + j * chunk, chunk)],
                            bufs.at[s])
            pend[s] = pltpu.async_copy(bufs.at[s], out_hbm.at[idx2.at[j]],
                                       wsems[s])
        for p in pend:
            if p is not None:
                p.wait()

    return sk(src, idx)


def _sc_gather(table, idx, n_rows, d, chunk):
    rpw = n_rows // _NW
    mesh = plsc.VectorSubcoreMesh(core_axis_name="c", subcore_axis_name="s")

    n = rpw // chunk

    @functools.partial(
        pl.kernel,
        mesh=mesh,
        out_type=jax.ShapeDtypeStruct((n_rows, d), jnp.float32),
        scratch_types=[
            pltpu.VMEM((n, chunk), jnp.int32),
            pltpu.VMEM((2, chunk, d), jnp.float32),
            pltpu.SemaphoreType.DMA,
            pltpu.SemaphoreType.DMA,
            pltpu.SemaphoreType.DMA,
        ],
    )
    def gk(table_hbm, idx_hbm, out_hbm, idx2, bufs, rsem, wsem0, wsem1):
        wid = lax.axis_index("s") * _SC_CORES + lax.axis_index("c")
        base = wid * rpw
        for j in range(n):
            pltpu.sync_copy(idx_hbm.at[pl.ds(base + j * chunk, chunk)],
                            idx2.at[j])
        wsems = (wsem0, wsem1)
        pend = [None, None]
        for j in range(n):
            s = j % 2
            if pend[s] is not None:
                pend[s].wait()
            pltpu.async_copy(table_hbm.at[idx2.at[j]], bufs.at[s],
                             rsem).wait()
            pend[s] = pltpu.async_copy(
                bufs.at[s], out_hbm.at[pl.ds(base + j * chunk, chunk)],
                wsems[s])
        for p in pend:
            if p is not None:
                p.wait()

    return gk(table, idx)


def _gmm_body(be_ref, rows_ref, w_ref, b_ref, o_ref, wb, last):
    g = pl.program_id(0)
    be = be_ref[g]
    is_real = be != NUM_T

    @pl.when(g == 0)
    def _():
        last[0] = -1

    @pl.when(is_real & (be != last[0]))
    def _():
        wb[...] = w_ref[0].astype(jnp.bfloat16)
        last[0] = be

    @pl.when(is_real)
    def _():
        r_lo, r_hi = _unpack_halves(rows_ref[...])
        acc = jnp.dot(r_lo, wb[:N1 // 2], preferred_element_type=jnp.float32)
        acc += jnp.dot(r_hi, wb[N1 // 2:], preferred_element_type=jnp.float32)
        out = jnp.maximum(acc + b_ref[0], 0.0)
        o_ref[...] = _pack_pairs(out.astype(jnp.bfloat16))

    @pl.when(jnp.logical_not(is_real))
    def _():
        o_ref[...] = jnp.zeros_like(o_ref)


def _gmm(bexp, rows, w, b):
    grid_spec = pltpu.PrefetchScalarGridSpec(
        num_scalar_prefetch=1,
        grid=(G,),
        in_specs=[
            pl.BlockSpec((BLK, N1 // 2),
                         lambda g, be: (jnp.where(be[g] == NUM_T, 0, g), 0)),
            pl.BlockSpec((1, N1, N2),
                         lambda g, be: (jnp.minimum(be[g], NUM_T - 1), 0, 0)),
            pl.BlockSpec((1, 1, N2),
                         lambda g, be: (jnp.minimum(be[g], NUM_T - 1), 0, 0)),
        ],
        out_specs=pl.BlockSpec((BLK, N2 // 2), lambda g, be: (g, 0)),
        scratch_shapes=[
            pltpu.VMEM((N1, N2), jnp.bfloat16),
            pltpu.SMEM((1,), jnp.int32),
        ],
    )
    return pl.pallas_call(
        _gmm_body,
        grid_spec=grid_spec,
        out_shape=jax.ShapeDtypeStruct((RP, N2 // 2), jnp.float32),
    )(bexp, rows, w, b)


def _dec_body(g0_ref, g1_ref, w1_ref, b1_ref, w2t_ref, b2_ref, o_ref):
    p0_lo, p0_hi = _unpack_halves(g0_ref[...])
    p1_lo, p1_hi = _unpack_halves(g1_ref[...])
    lat_lo = (p0_lo.astype(jnp.float32)
              + p1_lo.astype(jnp.float32)).astype(jnp.bfloat16)
    lat_hi = (p0_hi.astype(jnp.float32)
              + p1_hi.astype(jnp.float32)).astype(jnp.bfloat16)
    w1 = w1_ref[...].astype(jnp.bfloat16)
    d = jnp.dot(lat_lo, w1[:N2 // 2], preferred_element_type=jnp.float32)
    d += jnp.dot(lat_hi, w1[N2 // 2:], preferred_element_type=jnp.float32)
    d = jnp.maximum(d + b1_ref[...], 0.0)
    o_ref[...] = jnp.sum(d * w2t_ref[...], axis=1, keepdims=True) + b2_ref[...]


def _decoder(gath, w1, b1, w2t, b2):
    bm = 512
    nblk = B // bm
    return pl.pallas_call(
        _dec_body,
        grid=(nblk,),
        in_specs=[
            pl.BlockSpec((bm, N2 // 2), lambda i: (i, 0)),
            pl.BlockSpec((bm, N2 // 2), lambda i: (i + nblk, 0)),
            pl.BlockSpec((N2, N3), lambda i: (0, 0)),
            pl.BlockSpec((1, N3), lambda i: (0, 0)),
            pl.BlockSpec((1, N3), lambda i: (0, 0)),
            pl.BlockSpec((1, 1), lambda i: (0, 0)),
        ],
        out_specs=pl.BlockSpec((bm, 1), lambda i: (i, 0)),
        out_shape=jax.ShapeDtypeStruct((B, 1), jnp.float32),
    )(gath, gath, w1, b1, w2t, b2)


def kernel(input, treatment, enc_W1, enc_b1, enc_W2, enc_b2, exp_W, exp_b,
           dec_W1, dec_b1, dec_W2, dec_b2):
    t = treatment.astype(jnp.int32)
    t0, t1 = t[:, 0], t[:, 1]
    t1 = jnp.where(t1 == t0, NUM_T, t1)
    pair_e = jnp.concatenate([t0, t1])
    oh = (pair_e[None, :] == jnp.arange(NUM_T + 1, dtype=jnp.int32)[:, None]
          ).astype(jnp.int32)
    counts = jnp.sum(oh, axis=1)
    pos = jnp.sum((jnp.cumsum(oh, axis=1) - oh) * oh, axis=0)
    padded_counts = ((counts + BLK - 1) // BLK) * BLK
    ends = jnp.cumsum(padded_counts)
    starts = ends - padded_counts
    dest = starts[pair_e] + pos

    gate = counts[:NUM_T] > 1
    gate_pad = jnp.concatenate([gate, jnp.zeros((1,), bool)])
    bstart = jnp.arange(G, dtype=jnp.int32) * BLK
    bexp = jnp.searchsorted(ends, bstart, side="right").astype(jnp.int32)
    bexp = jnp.minimum(bexp, NUM_T)
    bexp = jnp.where(gate_pad[bexp], bexp, NUM_T)

    emb = _encoder(input, enc_W1, enc_b1.reshape(1, N0),
                   enc_W2, enc_b2.reshape(1, N1))

    rows = _sc_scatter_rows(emb, dest, RP, N1 // 2, chunk=64)

    partial = _gmm(bexp, rows, exp_W, exp_b.reshape(NUM_T, 1, N2))

    gath = _sc_gather(partial, dest, 2 * B, N2 // 2, chunk=128)

    out = _decoder(gath, dec_W1, dec_b1.reshape(1, N3),
                   dec_W2.reshape(1, N3), dec_b2.reshape(1, 1))
    return out


# speedup vs baseline: 1.0178x; 1.0178x over previous
import functools

import jax
import jax.numpy as jnp
from jax import lax
from jax.experimental import pallas as pl
from jax.experimental.pallas import tpu as pltpu
from jax.experimental.pallas import tpu_sc as plsc

B = 8192
IN_FEATURES = 2048
N0, N1, N2, N3 = 2048, 1024, 512, 1024
NUM_T = 16
BLK = 512
G = 2 * B // BLK + NUM_T + 1
RP = G * BLK

_SC_CORES = 2
_SC_SUBCORES = 16
_NW = _SC_CORES * _SC_SUBCORES


def _pack_pairs(x_bf16):
    n = x_bf16.shape[-1]
    lo = lax.bitcast_convert_type(x_bf16[:, :n // 2],
                                  jnp.uint16).astype(jnp.uint32)
    hi = lax.bitcast_convert_type(x_bf16[:, n // 2:],
                                  jnp.uint16).astype(jnp.uint32)
    return lax.bitcast_convert_type(lo | (hi << 16), jnp.float32)


def _unpack_halves(p_f32):
    u = lax.bitcast_convert_type(p_f32, jnp.uint32)
    lo = lax.bitcast_convert_type((u & 0xFFFF).astype(jnp.uint16),
                                  jnp.bfloat16)
    hi = lax.bitcast_convert_type((u >> 16).astype(jnp.uint16),
                                  jnp.bfloat16)
    return lo, hi


def _enc_body(x_ref, w1_ref, b1_ref, w2_ref, b2_ref, o_ref, w1b, w2b):
    @pl.when(pl.program_id(0) == 0)
    def _():
        w1b[...] = w1_ref[...].astype(jnp.bfloat16)
        w2b[...] = w2_ref[...].astype(jnp.bfloat16)

    x = x_ref[...].astype(jnp.bfloat16)
    h = jnp.dot(x, w1b[...], preferred_element_type=jnp.float32)
    h = jnp.maximum(h + b1_ref[...], 0.0).astype(jnp.bfloat16)
    e = jnp.dot(h, w2b[...], preferred_element_type=jnp.float32)
    e = jnp.maximum(e + b2_ref[...], 0.0)
    o_ref[...] = _pack_pairs(e.astype(jnp.bfloat16))


def _encoder(x, w1, b1, w2, b2):
    bm = 512
    return pl.pallas_call(
        _enc_body,
        grid=(B // bm,),
        in_specs=[
            pl.BlockSpec((bm, IN_FEATURES), lambda i: (i, 0)),
            pl.BlockSpec((IN_FEATURES, N0), lambda i: (0, 0)),
            pl.BlockSpec((1, N0), lambda i: (0, 0)),
            pl.BlockSpec((N0, N1), lambda i: (0, 0)),
            pl.BlockSpec((1, N1), lambda i: (0, 0)),
        ],
        out_specs=pl.BlockSpec((bm, N1 // 2), lambda i: (i, 0)),
        out_shape=jax.ShapeDtypeStruct((B, N1 // 2), jnp.float32),
        scratch_shapes=[
            pltpu.VMEM((IN_FEATURES, N0), jnp.bfloat16),
            pltpu.VMEM((N0, N1), jnp.bfloat16),
        ],
    )(x, w1, b1, w2, b2)


def _sc_scatter_rows(src, idx, n_out, d, chunk):
    n_pairs = idx.shape[0]
    src_rows = src.shape[0]
    rpw = n_pairs // _NW
    mesh = plsc.VectorSubcoreMesh(core_axis_name="c", subcore_axis_name="s")

    @functools.partial(
        pl.kernel,
        mesh=mesh,
        out_type=jax.ShapeDtypeStruct((n_out, d), jnp.float32),
        scratch_types=[
            pltpu.VMEM((chunk,), jnp.int32),
            pltpu.VMEM((chunk, d), jnp.float32),
            pltpu.SemaphoreType.DMA,
        ],
    )
    def sk(src_hbm, idx_hbm, out_hbm, idx_v, buf, sem):
        wid = lax.axis_index("s") * _SC_CORES + lax.axis_index("c")
        base = wid * rpw
        src_base = base % src_rows

        def body(j, carry):
            pltpu.sync_copy(idx_hbm.at[pl.ds(base + j * chunk, chunk)], idx_v)
            pltpu.sync_copy(src_hbm.at[pl.ds(src_base + j * chunk, chunk)],
                            buf)
            pltpu.async_copy(buf, out_hbm.at[idx_v], sem).wait()
            return carry

        lax.fori_loop(0, rpw // chunk, body, 0)

    return sk(src, idx)


def _sc_gather(table, idx, n_rows, d, chunk):
    rpw = n_rows // _NW
    mesh = plsc.VectorSubcoreMesh(core_axis_name="c", subcore_axis_name="s")

    @functools.partial(
        pl.kernel,
        mesh=mesh,
        out_type=jax.ShapeDtypeStruct((n_rows, d), jnp.float32),
        scratch_types=[
            pltpu.VMEM((chunk,), jnp.int32),
            pltpu.VMEM((chunk, d), jnp.float32),
            pltpu.SemaphoreType.DMA,
        ],
    )
    def gk(table_hbm, idx_hbm, out_hbm, idx_v, buf, sem):
        wid = lax.axis_index("s") * _SC_CORES + lax.axis_index("c")
        base = wid * rpw

        def body(j, carry):
            off = base + j * chunk
            pltpu.sync_copy(idx_hbm.at[pl.ds(off, chunk)], idx_v)
            pltpu.async_copy(table_hbm.at[idx_v], buf, sem).wait()
            pltpu.sync_copy(buf, out_hbm.at[pl.ds(off, chunk)])
            return carry

        lax.fori_loop(0, rpw // chunk, body, 0)

    return gk(table, idx)


def _gmm_body(be_ref, rows_ref, w_ref, b_ref, o_ref, wb, last):
    g = pl.program_id(0)
    be = be_ref[g]
    is_real = be != NUM_T

    @pl.when(g == 0)
    def _():
        last[0] = -1

    @pl.when(is_real & (be != last[0]))
    def _():
        wb[...] = w_ref[0].astype(jnp.bfloat16)
        last[0] = be

    @pl.when(is_real)
    def _():
        r_lo, r_hi = _unpack_halves(rows_ref[...])
        acc = jnp.dot(r_lo, wb[:N1 // 2], preferred_element_type=jnp.float32)
        acc += jnp.dot(r_hi, wb[N1 // 2:], preferred_element_type=jnp.float32)
        out = jnp.maximum(acc + b_ref[0], 0.0)
        o_ref[...] = _pack_pairs(out.astype(jnp.bfloat16))

    @pl.when(jnp.logical_not(is_real))
    def _():
        o_ref[...] = jnp.zeros_like(o_ref)


def _gmm(bexp, rows, w, b):
    grid_spec = pltpu.PrefetchScalarGridSpec(
        num_scalar_prefetch=1,
        grid=(G,),
        in_specs=[
            pl.BlockSpec((BLK, N1 // 2),
                         lambda g, be: (jnp.where(be[g] == NUM_T, 0, g), 0)),
            pl.BlockSpec((1, N1, N2),
                         lambda g, be: (jnp.minimum(be[g], NUM_T - 1), 0, 0)),
            pl.BlockSpec((1, 1, N2),
                         lambda g, be: (jnp.minimum(be[g], NUM_T - 1), 0, 0)),
        ],
        out_specs=pl.BlockSpec((BLK, N2 // 2), lambda g, be: (g, 0)),
        scratch_shapes=[
            pltpu.VMEM((N1, N2), jnp.bfloat16),
            pltpu.SMEM((1,), jnp.int32),
        ],
    )
    return pl.pallas_call(
        _gmm_body,
        grid_spec=grid_spec,
        out_shape=jax.ShapeDtypeStruct((RP, N2 // 2), jnp.float32),
    )(bexp, rows, w, b)


def _dec_body(g0_ref, g1_ref, w1_ref, b1_ref, w2t_ref, b2_ref, o_ref):
    p0_lo, p0_hi = _unpack_halves(g0_ref[...])
    p1_lo, p1_hi = _unpack_halves(g1_ref[...])
    lat_lo = (p0_lo.astype(jnp.float32)
              + p1_lo.astype(jnp.float32)).astype(jnp.bfloat16)
    lat_hi = (p0_hi.astype(jnp.float32)
              + p1_hi.astype(jnp.float32)).astype(jnp.bfloat16)
    w1 = w1_ref[...].astype(jnp.bfloat16)
    d = jnp.dot(lat_lo, w1[:N2 // 2], preferred_element_type=jnp.float32)
    d += jnp.dot(lat_hi, w1[N2 // 2:], preferred_element_type=jnp.float32)
    d = jnp.maximum(d + b1_ref[...], 0.0)
    o_ref[...] = jnp.sum(d * w2t_ref[...], axis=1, keepdims=True) + b2_ref[...]


def _decoder(gath, w1, b1, w2t, b2):
    bm = 512
    nblk = B // bm
    return pl.pallas_call(
        _dec_body,
        grid=(nblk,),
        in_specs=[
            pl.BlockSpec((bm, N2 // 2), lambda i: (i, 0)),
            pl.BlockSpec((bm, N2 // 2), lambda i: (i + nblk, 0)),
            pl.BlockSpec((N2, N3), lambda i: (0, 0)),
            pl.BlockSpec((1, N3), lambda i: (0, 0)),
            pl.BlockSpec((1, N3), lambda i: (0, 0)),
            pl.BlockSpec((1, 1), lambda i: (0, 0)),
        ],
        out_specs=pl.BlockSpec((bm, 1), lambda i: (i, 0)),
        out_shape=jax.ShapeDtypeStruct((B, 1), jnp.float32),
    )(gath, gath, w1, b1, w2t, b2)


def kernel(input, treatment, enc_W1, enc_b1, enc_W2, enc_b2, exp_W, exp_b,
           dec_W1, dec_b1, dec_W2, dec_b2):
    t = treatment.astype(jnp.int32)
    t0, t1 = t[:, 0], t[:, 1]
    t1 = jnp.where(t1 == t0, NUM_T, t1)
    pair_e = jnp.concatenate([t0, t1])
    oh = (pair_e[None, :] == jnp.arange(NUM_T + 1, dtype=jnp.int32)[:, None]
          ).astype(jnp.int32)
    counts = jnp.sum(oh, axis=1)
    pos = jnp.sum((jnp.cumsum(oh, axis=1) - oh) * oh, axis=0)
    padded_counts = ((counts + BLK - 1) // BLK) * BLK
    ends = jnp.cumsum(padded_counts)
    starts = ends - padded_counts
    dest = starts[pair_e] + pos

    gate = counts[:NUM_T] > 1
    gate_pad = jnp.concatenate([gate, jnp.zeros((1,), bool)])
    bstart = jnp.arange(G, dtype=jnp.int32) * BLK
    bexp = jnp.searchsorted(ends, bstart, side="right").astype(jnp.int32)
    bexp = jnp.minimum(bexp, NUM_T)
    bexp = jnp.where(gate_pad[bexp], bexp, NUM_T)

    emb = _encoder(input, enc_W1, enc_b1.reshape(1, N0),
                   enc_W2, enc_b2.reshape(1, N1))

    rows = _sc_scatter_rows(emb, dest, RP, N1 // 2, chunk=128)

    partial = _gmm(bexp, rows, exp_W, exp_b.reshape(NUM_T, 1, N2))

    gath = _sc_gather(partial, dest, 2 * B, N2 // 2, chunk=128)

    out = _decoder(gath, dec_W1, dec_b1.reshape(1, N3),
                   dec_W2.reshape(1, N3), dec_b2.reshape(1, 1))
    return out


# speedup vs baseline: 1.0612x; 1.0427x over previous
import functools

import jax
import jax.numpy as jnp
from jax import lax
from jax.experimental import pallas as pl
from jax.experimental.pallas import tpu as pltpu
from jax.experimental.pallas import tpu_sc as plsc

B = 8192
IN_FEATURES = 2048
N0, N1, N2, N3 = 2048, 1024, 512, 1024
NUM_T = 16
BLK = 1024
G = 2 * B // BLK + NUM_T + 1
RP = G * BLK

_SC_CORES = 2
_SC_SUBCORES = 16
_NW = _SC_CORES * _SC_SUBCORES


def _pack_pairs(x_bf16):
    n = x_bf16.shape[-1]
    lo = lax.bitcast_convert_type(x_bf16[:, :n // 2],
                                  jnp.uint16).astype(jnp.uint32)
    hi = lax.bitcast_convert_type(x_bf16[:, n // 2:],
                                  jnp.uint16).astype(jnp.uint32)
    return lax.bitcast_convert_type(lo | (hi << 16), jnp.float32)


def _unpack_halves(p_f32):
    u = lax.bitcast_convert_type(p_f32, jnp.uint32)
    lo = lax.bitcast_convert_type((u & 0xFFFF).astype(jnp.uint16),
                                  jnp.bfloat16)
    hi = lax.bitcast_convert_type((u >> 16).astype(jnp.uint16),
                                  jnp.bfloat16)
    return lo, hi


def _enc_body(x_ref, w1_ref, b1_ref, w2_ref, b2_ref, o_ref, w1b, w2b):
    @pl.when(pl.program_id(0) == 0)
    def _():
        w1b[...] = w1_ref[...].astype(jnp.bfloat16)
        w2b[...] = w2_ref[...].astype(jnp.bfloat16)

    x = x_ref[...].astype(jnp.bfloat16)
    h = jnp.dot(x, w1b[...], preferred_element_type=jnp.float32)
    h = jnp.maximum(h + b1_ref[...], 0.0).astype(jnp.bfloat16)
    e = jnp.dot(h, w2b[...], preferred_element_type=jnp.float32)
    e = jnp.maximum(e + b2_ref[...], 0.0)
    o_ref[...] = _pack_pairs(e.astype(jnp.bfloat16))


def _encoder(x, w1, b1, w2, b2):
    bm = 512
    return pl.pallas_call(
        _enc_body,
        grid=(B // bm,),
        in_specs=[
            pl.BlockSpec((bm, IN_FEATURES), lambda i: (i, 0)),
            pl.BlockSpec((IN_FEATURES, N0), lambda i: (0, 0)),
            pl.BlockSpec((1, N0), lambda i: (0, 0)),
            pl.BlockSpec((N0, N1), lambda i: (0, 0)),
            pl.BlockSpec((1, N1), lambda i: (0, 0)),
        ],
        out_specs=pl.BlockSpec((bm, N1 // 2), lambda i: (i, 0)),
        out_shape=jax.ShapeDtypeStruct((B, N1 // 2), jnp.float32),
        scratch_shapes=[
            pltpu.VMEM((IN_FEATURES, N0), jnp.bfloat16),
            pltpu.VMEM((N0, N1), jnp.bfloat16),
        ],
    )(x, w1, b1, w2, b2)


def _sc_scatter_rows(src, idx, n_out, d, chunk):
    n_pairs = idx.shape[0]
    src_rows = src.shape[0]
    rpw = n_pairs // _NW
    mesh = plsc.VectorSubcoreMesh(core_axis_name="c", subcore_axis_name="s")

    @functools.partial(
        pl.kernel,
        mesh=mesh,
        out_type=jax.ShapeDtypeStruct((n_out, d), jnp.float32),
        scratch_types=[
            pltpu.VMEM((chunk,), jnp.int32),
            pltpu.VMEM((chunk, d), jnp.float32),
            pltpu.SemaphoreType.DMA,
        ],
    )
    def sk(src_hbm, idx_hbm, out_hbm, idx_v, buf, sem):
        wid = lax.axis_index("s") * _SC_CORES + lax.axis_index("c")
        base = wid * rpw
        src_base = base % src_rows

        def body(j, carry):
            pltpu.sync_copy(idx_hbm.at[pl.ds(base + j * chunk, chunk)], idx_v)
            pltpu.sync_copy(src_hbm.at[pl.ds(src_base + j * chunk, chunk)],
                            buf)
            pltpu.async_copy(buf, out_hbm.at[idx_v], sem).wait()
            return carry

        lax.fori_loop(0, rpw // chunk, body, 0)

    return sk(src, idx)


def _sc_gather(table, idx, n_rows, d, chunk):
    rpw = n_rows // _NW
    mesh = plsc.VectorSubcoreMesh(core_axis_name="c", subcore_axis_name="s")

    @functools.partial(
        pl.kernel,
        mesh=mesh,
        out_type=jax.ShapeDtypeStruct((n_rows, d), jnp.float32),
        scratch_types=[
            pltpu.VMEM((chunk,), jnp.int32),
            pltpu.VMEM((chunk, d), jnp.float32),
            pltpu.SemaphoreType.DMA,
        ],
    )
    def gk(table_hbm, idx_hbm, out_hbm, idx_v, buf, sem):
        wid = lax.axis_index("s") * _SC_CORES + lax.axis_index("c")
        base = wid * rpw

        def body(j, carry):
            off = base + j * chunk
            pltpu.sync_copy(idx_hbm.at[pl.ds(off, chunk)], idx_v)
            pltpu.async_copy(table_hbm.at[idx_v], buf, sem).wait()
            pltpu.sync_copy(buf, out_hbm.at[pl.ds(off, chunk)])
            return carry

        lax.fori_loop(0, rpw // chunk, body, 0)

    return gk(table, idx)


def _gmm_body(be_ref, rows_ref, w_ref, b_ref, o_ref, wb, last):
    g = pl.program_id(0)
    be = be_ref[g]
    is_real = be != NUM_T

    @pl.when(g == 0)
    def _():
        last[0] = -1

    @pl.when(is_real & (be != last[0]))
    def _():
        wb[...] = w_ref[0].astype(jnp.bfloat16)
        last[0] = be

    @pl.when(is_real)
    def _():
        r_lo, r_hi = _unpack_halves(rows_ref[...])
        acc = jnp.dot(r_lo, wb[:N1 // 2], preferred_element_type=jnp.float32)
        acc += jnp.dot(r_hi, wb[N1 // 2:], preferred_element_type=jnp.float32)
        out = jnp.maximum(acc + b_ref[0], 0.0)
        o_ref[...] = _pack_pairs(out.astype(jnp.bfloat16))

    @pl.when(jnp.logical_not(is_real))
    def _():
        o_ref[...] = jnp.zeros_like(o_ref)


def _gmm(bexp, rows, w, b):
    grid_spec = pltpu.PrefetchScalarGridSpec(
        num_scalar_prefetch=1,
        grid=(G,),
        in_specs=[
            pl.BlockSpec((BLK, N1 // 2),
                         lambda g, be: (jnp.where(be[g] == NUM_T, 0, g), 0)),
            pl.BlockSpec((1, N1, N2),
                         lambda g, be: (jnp.minimum(be[g], NUM_T - 1), 0, 0)),
            pl.BlockSpec((1, 1, N2),
                         lambda g, be: (jnp.minimum(be[g], NUM_T - 1), 0, 0)),
        ],
        out_specs=pl.BlockSpec((BLK, N2 // 2), lambda g, be: (g, 0)),
        scratch_shapes=[
            pltpu.VMEM((N1, N2), jnp.bfloat16),
            pltpu.SMEM((1,), jnp.int32),
        ],
    )
    return pl.pallas_call(
        _gmm_body,
        grid_spec=grid_spec,
        out_shape=jax.ShapeDtypeStruct((RP, N2 // 2), jnp.float32),
    )(bexp, rows, w, b)


def _dec_body(g0_ref, g1_ref, w1_ref, b1_ref, w2t_ref, b2_ref, o_ref):
    p0_lo, p0_hi = _unpack_halves(g0_ref[...])
    p1_lo, p1_hi = _unpack_halves(g1_ref[...])
    lat_lo = (p0_lo.astype(jnp.float32)
              + p1_lo.astype(jnp.float32)).astype(jnp.bfloat16)
    lat_hi = (p0_hi.astype(jnp.float32)
              + p1_hi.astype(jnp.float32)).astype(jnp.bfloat16)
    w1 = w1_ref[...].astype(jnp.bfloat16)
    d = jnp.dot(lat_lo, w1[:N2 // 2], preferred_element_type=jnp.float32)
    d += jnp.dot(lat_hi, w1[N2 // 2:], preferred_element_type=jnp.float32)
    d = jnp.maximum(d + b1_ref[...], 0.0)
    o_ref[...] = jnp.sum(d * w2t_ref[...], axis=1, keepdims=True) + b2_ref[...]


def _decoder(gath, w1, b1, w2t, b2):
    bm = 512
    nblk = B // bm
    return pl.pallas_call(
        _dec_body,
        grid=(nblk,),
        in_specs=[
            pl.BlockSpec((bm, N2 // 2), lambda i: (i, 0)),
            pl.BlockSpec((bm, N2 // 2), lambda i: (i + nblk, 0)),
            pl.BlockSpec((N2, N3), lambda i: (0, 0)),
            pl.BlockSpec((1, N3), lambda i: (0, 0)),
            pl.BlockSpec((1, N3), lambda i: (0, 0)),
            pl.BlockSpec((1, 1), lambda i: (0, 0)),
        ],
        out_specs=pl.BlockSpec((bm, 1), lambda i: (i, 0)),
        out_shape=jax.ShapeDtypeStruct((B, 1), jnp.float32),
    )(gath, gath, w1, b1, w2t, b2)


def kernel(input, treatment, enc_W1, enc_b1, enc_W2, enc_b2, exp_W, exp_b,
           dec_W1, dec_b1, dec_W2, dec_b2):
    t = treatment.astype(jnp.int32)
    t0, t1 = t[:, 0], t[:, 1]
    t1 = jnp.where(t1 == t0, NUM_T, t1)
    pair_e = jnp.concatenate([t0, t1])
    oh = (pair_e[None, :] == jnp.arange(NUM_T + 1, dtype=jnp.int32)[:, None]
          ).astype(jnp.int32)
    counts = jnp.sum(oh, axis=1)
    pos = jnp.sum((jnp.cumsum(oh, axis=1) - oh) * oh, axis=0)
    padded_counts = ((counts + BLK - 1) // BLK) * BLK
    ends = jnp.cumsum(padded_counts)
    starts = ends - padded_counts
    dest = starts[pair_e] + pos

    gate = counts[:NUM_T] > 1
    gate_pad = jnp.concatenate([gate, jnp.zeros((1,), bool)])
    bstart = jnp.arange(G, dtype=jnp.int32) * BLK
    bexp = jnp.searchsorted(ends, bstart, side="right").astype(jnp.int32)
    bexp = jnp.minimum(bexp, NUM_T)
    bexp = jnp.where(gate_pad[bexp], bexp, NUM_T)

    emb = _encoder(input, enc_W1, enc_b1.reshape(1, N0),
                   enc_W2, enc_b2.reshape(1, N1))

    rows = _sc_scatter_rows(emb, dest, RP, N1 // 2, chunk=128)

    partial = _gmm(bexp, rows, exp_W, exp_b.reshape(NUM_T, 1, N2))

    gath = _sc_gather(partial, dest, 2 * B, N2 // 2, chunk=128)

    out = _decoder(gath, dec_W1, dec_b1.reshape(1, N3),
                   dec_W2.reshape(1, N3), dec_b2.reshape(1, 1))
    return out
